# baseline (speedup 1.0000x reference)
import jax
import jax.numpy as jnp
from jax import lax
from jax.experimental import pallas as pl
from jax.experimental.pallas import tpu as pltpu

N = 4096
D = 4096
E = 64
TOPK = 2
CAP = 128

TM = 512
TB = 128


def _mm_rows_kernel(x_ref, w_ref, keys_ref, gsig_ref):
    logits = lax.dot_general(
        x_ref[:], w_ref[:], (((1,), (1,)), ((), ())),
        preferred_element_type=jnp.float32)
    col = lax.broadcasted_iota(jnp.int32, (TM, E), 1)

    m1 = jnp.max(logits, axis=1, keepdims=True)
    i1 = jnp.min(jnp.where(logits == m1, col, E), axis=1, keepdims=True)
    is1 = col == i1
    l2 = jnp.where(is1, -jnp.inf, logits)
    m2 = jnp.max(l2, axis=1, keepdims=True)
    i2 = jnp.min(jnp.where(l2 == m2, col, E), axis=1, keepdims=True)
    mask = is1 | (col == i2)

    z = jnp.exp(logits - m1)
    gates = z / jnp.sum(z, axis=1, keepdims=True)
    gsig_ref[:] = jnp.where(mask, -gates, gates)

    v = jnp.where(mask, logits, 0.0)
    vi = lax.bitcast_convert_type(v, jnp.int32)
    keys_ref[:] = vi ^ jnp.where(vi < 0, jnp.int32(0x7FFFFFFF), jnp.int32(0))


def _cumsum0(a):
    n, m = a.shape
    s = 1
    while s < n:
        shifted = jnp.concatenate(
            [jnp.zeros((s, m), a.dtype), a[:-s]], axis=0)
        a = a + shifted
        s *= 2
    return a


def _route_mat_kernel(keys_ref, gsig_ref, comb_ref, disp_ref,
                      laux_ref, cnt_ref, gm_s, loc_s):
    i = pl.program_id(0)

    @pl.when(i == 0)
    def _route():
        keys = keys_ref[:]
        gsig = gsig_ref[:]
        mask = gsig < 0.0
        gates = jnp.abs(gsig)

        cnt = jnp.sum(mask.astype(jnp.int32), axis=0, keepdims=True)
        cnt_ref[:] = cnt
        me = jnp.sum(gates, axis=0, keepdims=True) * (1.0 / N)
        ce = cnt.astype(jnp.float32) * (1.0 / N)
        laux_ref[:] = jnp.sum(me * ce, axis=1, keepdims=True) * (float(E) / TOPK)

        keys2 = jnp.concatenate([keys[: N // 2], keys[N // 2:]], axis=1)

        def _colcount(m2):
            s = jnp.sum(m2.astype(jnp.int32), axis=0, keepdims=True)
            return s[:, :E] + s[:, E:]

        nonneg = _colcount(keys2 >= 0)
        int_min = jnp.int32(-2147483647 - 1)
        theta0 = jnp.where(nonneg >= CAP, jnp.int32(0), int_min)

        def body(it, theta):
            b = 30 - it
            cand = theta | lax.shift_left(jnp.int32(1), b)
            cand2 = jnp.concatenate([cand, cand], axis=1)
            ge = _colcount(keys2 >= cand2)
            return jnp.where(ge >= CAP, cand, theta)

        theta = lax.fori_loop(0, 31, body, theta0)

        theta2 = jnp.concatenate([theta, theta], axis=1)
        n_gt = _colcount(keys2 > theta2)
        is_eq = keys == theta
        eq_i = is_eq.astype(jnp.int32)
        eq_rank = _cumsum0(eq_i) - eq_i
        keep = (keys > theta) | (is_eq & (eq_rank < (CAP - n_gt)))
        fmask = mask & keep

        fi = fmask.astype(jnp.int32)
        loc = _cumsum0(fi) - 1

        gm = gates * fi.astype(jnp.float32)
        denom = jnp.maximum(jnp.sum(gm, axis=1, keepdims=True),
                            jnp.float32(1.1920929e-07))
        gm_s[:] = gm / denom
        loc_s[:] = jnp.where(fmask, loc, 0)

    @pl.when(i > 0)
    def _materialize():
        t0 = (i - 1) * TB
        gm = gm_s[pl.ds(t0, TB), :][:, :, None]
        loc = loc_s[pl.ds(t0, TB), :][:, :, None]
        cid = lax.broadcasted_iota(jnp.int32, (TB, E, CAP), 2)
        comb = jnp.where(cid == loc, gm, 0.0)
        comb_ref[:] = comb
        disp_ref[:] = (comb > 0.0).astype(jnp.int8)


def kernel(input, W):
    x = input.astype(jnp.float32)
    w = W.astype(jnp.float32)

    keys, gsig = pl.pallas_call(
        _mm_rows_kernel,
        grid=(N // TM,),
        in_specs=[pl.BlockSpec((TM, D), lambda i: (i, 0)),
                  pl.BlockSpec((E, D), lambda i: (0, 0))],
        out_specs=(pl.BlockSpec((TM, E), lambda i: (i, 0)),
                   pl.BlockSpec((TM, E), lambda i: (i, 0))),
        out_shape=(jax.ShapeDtypeStruct((N, E), jnp.int32),
                   jax.ShapeDtypeStruct((N, E), jnp.float32)),
    )(x, w)

    comb, disp, laux, cnt = pl.pallas_call(
        _route_mat_kernel,
        grid=(1 + N // TB,),
        in_specs=[pl.BlockSpec((N, E), lambda i: (0, 0)),
                  pl.BlockSpec((N, E), lambda i: (0, 0))],
        out_specs=(
            pl.BlockSpec((TB, E, CAP), lambda i: (jnp.maximum(i - 1, 0), 0, 0)),
            pl.BlockSpec((TB, E, CAP), lambda i: (jnp.maximum(i - 1, 0), 0, 0)),
            pl.BlockSpec((1, 1), lambda i: (0, 0)),
            pl.BlockSpec((1, E), lambda i: (0, 0))),
        out_shape=(jax.ShapeDtypeStruct((N, E, CAP), jnp.float32),
                   jax.ShapeDtypeStruct((N, E, CAP), jnp.int8),
                   jax.ShapeDtypeStruct((1, 1), jnp.float32),
                   jax.ShapeDtypeStruct((1, E), jnp.int32)),
        scratch_shapes=[pltpu.VMEM((N, E), jnp.float32),
                        pltpu.VMEM((N, E), jnp.int32)],
    )(keys, gsig)

    return laux[0, 0], comb, disp.astype(jnp.bool_), cnt[0]


# speedup vs baseline: 1.0260x; 1.0260x over previous
import jax
import jax.numpy as jnp
from jax import lax
from jax.experimental import pallas as pl
from jax.experimental.pallas import tpu as pltpu

N = 4096
D = 4096
E = 64
TOPK = 2
CAP = 128

TM = 512
TB = 256
NMM = N // TM
ROUTE = NMM
NSTEPS = NMM + 1 + N // TB


def _cumsum0(a):
    n, m = a.shape
    s = 1
    while s < n:
        shifted = jnp.concatenate(
            [jnp.zeros((s, m), a.dtype), a[:-s]], axis=0)
        a = a + shifted
        s *= 2
    return a


def _fused_kernel(x_ref, w_ref, comb_ref, disp_ref, laux_ref, cnt_ref,
                  keys_s, gsig_s, gm_s, loc_s):
    i = pl.program_id(0)

    @pl.when(i < ROUTE)
    def _mm_rows():
        logits = lax.dot_general(
            x_ref[:], w_ref[:], (((1,), (1,)), ((), ())),
            preferred_element_type=jnp.float32)
        col = lax.broadcasted_iota(jnp.int32, (TM, E), 1)

        m1 = jnp.max(logits, axis=1, keepdims=True)
        i1 = jnp.min(jnp.where(logits == m1, col, E), axis=1, keepdims=True)
        is1 = col == i1
        l2 = jnp.where(is1, -jnp.inf, logits)
        m2 = jnp.max(l2, axis=1, keepdims=True)
        i2 = jnp.min(jnp.where(l2 == m2, col, E), axis=1, keepdims=True)
        mask = is1 | (col == i2)

        z = jnp.exp(logits - m1)
        gates = z / jnp.sum(z, axis=1, keepdims=True)
        t0 = i * TM
        gsig_s[pl.ds(t0, TM), :] = jnp.where(mask, -gates, gates)

        v = jnp.where(mask, logits, 0.0)
        vi = lax.bitcast_convert_type(v, jnp.int32)
        keys_s[pl.ds(t0, TM), :] = (
            vi ^ jnp.where(vi < 0, jnp.int32(0x7FFFFFFF), jnp.int32(0)))

    @pl.when(i == ROUTE)
    def _route():
        keys = keys_s[:]
        gsig = gsig_s[:]
        mask = gsig < 0.0
        gates = jnp.abs(gsig)

        cnt = jnp.sum(mask.astype(jnp.int32), axis=0, keepdims=True)
        cnt_ref[:] = cnt
        me = jnp.sum(gates, axis=0, keepdims=True) * (1.0 / N)
        ce = cnt.astype(jnp.float32) * (1.0 / N)
        laux_ref[:] = jnp.sum(me * ce, axis=1, keepdims=True) * (float(E) / TOPK)

        keys2 = jnp.concatenate([keys[: N // 2], keys[N // 2:]], axis=1)

        def _colcount(m2):
            s = jnp.sum(m2.astype(jnp.int32), axis=0, keepdims=True)
            return s[:, :E] + s[:, E:]

        nonneg = _colcount(keys2 >= 0)
        int_min = jnp.int32(-2147483647 - 1)
        theta0 = jnp.where(nonneg >= CAP, jnp.int32(0), int_min)

        def body(it, theta):
            b = 30 - it
            cand = theta | lax.shift_left(jnp.int32(1), b)
            cand2 = jnp.concatenate([cand, cand], axis=1)
            ge = _colcount(keys2 >= cand2)
            return jnp.where(ge >= CAP, cand, theta)

        theta = lax.fori_loop(0, 31, body, theta0)

        theta2 = jnp.concatenate([theta, theta], axis=1)
        n_gt = _colcount(keys2 > theta2)
        is_eq = keys == theta
        eq_i = is_eq.astype(jnp.int32)
        eq_rank = _cumsum0(eq_i) - eq_i
        keep = (keys > theta) | (is_eq & (eq_rank < (CAP - n_gt)))
        fmask = mask & keep

        fi = fmask.astype(jnp.int32)
        loc = _cumsum0(fi) - 1

        gm = gates * fi.astype(jnp.float32)
        denom = jnp.maximum(jnp.sum(gm, axis=1, keepdims=True),
                            jnp.float32(1.1920929e-07))
        gm_s[:] = gm / denom
        loc_s[:] = jnp.where(fmask, loc, 0)

    @pl.when(i > ROUTE)
    def _materialize():
        t0 = (i - ROUTE - 1) * TB
        gm = gm_s[pl.ds(t0, TB), :][:, :, None]
        loc = loc_s[pl.ds(t0, TB), :][:, :, None]
        cid = lax.broadcasted_iota(jnp.int32, (TB, E, CAP), 2)
        comb = jnp.where(cid == loc, gm, 0.0)
        comb_ref[:] = comb
        disp_ref[:] = (comb > 0.0).astype(jnp.int8)


def kernel(input, W):
    x = input.astype(jnp.float32)
    w = W.astype(jnp.float32)

    comb, disp, laux, cnt = pl.pallas_call(
        _fused_kernel,
        grid=(NSTEPS,),
        in_specs=[
            pl.BlockSpec((TM, D), lambda i: (jnp.minimum(i, NMM - 1), 0)),
            pl.BlockSpec((E, D), lambda i: (0, 0)),
        ],
        out_specs=(
            pl.BlockSpec((TB, E, CAP),
                         lambda i: (jnp.maximum(i - ROUTE - 1, 0), 0, 0)),
            pl.BlockSpec((TB, E, CAP),
                         lambda i: (jnp.maximum(i - ROUTE - 1, 0), 0, 0)),
            pl.BlockSpec((1, 1), lambda i: (0, 0)),
            pl.BlockSpec((1, E), lambda i: (0, 0))),
        out_shape=(jax.ShapeDtypeStruct((N, E, CAP), jnp.float32),
                   jax.ShapeDtypeStruct((N, E, CAP), jnp.int8),
                   jax.ShapeDtypeStruct((1, 1), jnp.float32),
                   jax.ShapeDtypeStruct((1, E), jnp.int32)),
        scratch_shapes=[pltpu.VMEM((N, E), jnp.int32),
                        pltpu.VMEM((N, E), jnp.float32),
                        pltpu.VMEM((N, E), jnp.float32),
                        pltpu.VMEM((N, E), jnp.int32)],
    )(x, w)

    return laux[0, 0], comb, disp.astype(jnp.bool_), cnt[0]
